# baseline (speedup 1.0000x reference)
import functools

import jax
import jax.numpy as jnp
from jax import lax
from jax.experimental import pallas as pl
from jax.experimental.pallas import tpu as pltpu
from jax.experimental.pallas import tpu_sc as plsc

_B, _D, _E = 8192, 4096, 64
_H1, _H2, _H3 = 128, 256, 128
_R = 1024
_N = _B // _R

_NS = 16
_RT = _B // _NS
_L = 16

_DN = (((1,), (1,)), ((), ()))


def _mlp_kernel(x_ref, w1_ref, b1_ref, w2_ref, b2_ref, w3_ref, b3_ref,
                w4_ref, b4_ref, out_ref):
    h = jax.lax.dot_general(x_ref[...], w1_ref[...], _DN,
                            preferred_element_type=jnp.float32) + b1_ref[...]
    h = jnp.maximum(h, 0.0)
    h = jax.lax.dot_general(h, w2_ref[...], _DN,
                            preferred_element_type=jnp.float32) + b2_ref[...]
    h = jnp.where(h >= 0, h, 0.01 * h)
    h = jax.lax.dot_general(h, w3_ref[...], _DN,
                            preferred_element_type=jnp.float32) + b3_ref[...]
    h = jnp.where(h >= 0, h, 0.01 * h)
    out_ref[...] = jax.lax.dot_general(h, w4_ref[...], _DN,
                                       preferred_element_type=jnp.float32) + b4_ref[...]


def _mlp_logits(x, W1, b1r, W2, b2r, W3, b3r, W4, b4r):
    blk = lambda i: (i, 0)
    fixed = lambda i: (0, 0)
    return pl.pallas_call(
        _mlp_kernel,
        grid=(_N,),
        in_specs=[
            pl.BlockSpec((_R, _D), blk),
            pl.BlockSpec((_H1, _D), fixed),
            pl.BlockSpec((1, _H1), fixed),
            pl.BlockSpec((_H2, _H1), fixed),
            pl.BlockSpec((1, _H2), fixed),
            pl.BlockSpec((_H3, _H2), fixed),
            pl.BlockSpec((1, _H3), fixed),
            pl.BlockSpec((_E, _H3), fixed),
            pl.BlockSpec((1, _E), fixed),
        ],
        out_specs=pl.BlockSpec((_R, _E), blk),
        out_shape=jax.ShapeDtypeStruct((_B, _E), jnp.float32),
    )(x, W1, b1r, W2, b2r, W3, b3r, W4, b4r)


_NEG = float("-inf")


_TW = _RT * _E


def _route_body(logits_hbm, out_hbm, lvm, ovm, part_vm, parts_vm, shared):
    sid = lax.axis_index("s")
    base = sid * _TW
    lane = lax.broadcasted_iota(jnp.int32, (_L,), 0)

    zero = jnp.zeros((_L,), jnp.float32)

    def zrow(r, t):
        ovm[pl.ds(r * _L, _L)] = zero
        return t

    lax.fori_loop(0, _TW // _L, zrow, 0)

    _CH = _TW // 4
    for cc in range(4):
        pltpu.sync_copy(logits_hbm.at[pl.ds(base + cc * _CH, _CH)],
                        lvm.at[pl.ds(cc * _CH, _CH)])

    def group(g, acc):
        start = (lane + g * _L) * _E
        m1 = jnp.full((_L,), _NEG, jnp.float32)
        m2 = jnp.full((_L,), _NEG, jnp.float32)
        for j in range(_E):
            v = plsc.load_gather(lvm, [start + j])
            nm1 = jnp.maximum(m1, v)
            m2 = jnp.maximum(m2, jnp.minimum(m1, v))
            m1 = nm1
        return acc + m1 + m2

    acc = lax.fori_loop(0, _RT // _L, group, jnp.zeros((_L,), jnp.float32))

    part_vm[...] = acc
    pltpu.sync_copy(part_vm, shared.at[sid])

    plsc.subcore_barrier()

    @pl.when(sid == 0)
    def _row0():
        pltpu.sync_copy(shared, parts_vm)

        def addp(t, tot):
            return tot + parts_vm[t, :]

        tot = lax.fori_loop(0, _NS, addp, jnp.zeros((_L,), jnp.float32))
        s = jnp.sum(tot)

        m1 = lvm[pl.ds(0, _L)]
        i1 = lane
        m2 = jnp.full((_L,), _NEG, jnp.float32)
        i2 = jnp.full((_L,), _E, jnp.int32)
        for k in range(1, _E // _L):
            v = lvm[pl.ds(k * _L, _L)]
            cols = lane + k * _L
            c1 = v > m1
            c2 = jnp.logical_and(jnp.logical_not(c1), v > m2)
            m2 = jnp.where(c1, m1, jnp.where(c2, v, m2))
            i2 = jnp.where(c1, i1, jnp.where(c2, cols, i2))
            m1 = jnp.where(c1, v, m1)
            i1 = jnp.where(c1, cols, i1)
        big = jnp.int32(2 * _E)
        v1 = jnp.max(m1)
        j1 = jnp.min(jnp.where(m1 == v1, i1, big))
        lstar = jnp.remainder(j1, _L)
        cand = jnp.where(lane == lstar, m2, m1)
        cidx = jnp.where(lane == lstar, i2, i1)
        v2 = jnp.max(cand)
        j2 = jnp.min(jnp.where(cand == v2, cidx, big))

        sv = s * jnp.ones((_L,), jnp.float32)
        r = plsc.bitcast(jnp.int32(0x7EF311C3) - plsc.bitcast(sv, jnp.int32),
                         jnp.float32)
        for _ in range(3):
            r = r * (2.0 - sv * r)
        w1v = v1 * r
        w2v = v2 * r
        for k in range(_E // _L):
            cols = lane + k * _L
            vals = (jnp.where(cols == j1, w1v, 0.0)
                    + jnp.where(cols == j2, w2v, 0.0))
            ovm[pl.ds(k * _L, _L)] = vals

    pltpu.sync_copy(ovm, out_hbm.at[pl.ds(base, _TW)])


_route = functools.partial(
    pl.kernel,
    out_type=jax.ShapeDtypeStruct((_B * _E,), jnp.float32),
    mesh=plsc.VectorSubcoreMesh(core_axis_name="c", subcore_axis_name="s",
                                num_cores=1),
    compiler_params=pltpu.CompilerParams(needs_layout_passes=False),
    scratch_types=[
        pltpu.VMEM((_TW,), jnp.float32),
        pltpu.VMEM((_TW,), jnp.float32),
        pltpu.VMEM((_L,), jnp.float32),
        pltpu.VMEM((_NS, _L), jnp.float32),
        pltpu.VMEM_SHARED((_NS, _L), jnp.float32),
    ],
)(_route_body)


def kernel(x, W1, b1, W2, b2, W3, b3, W4, b4):
    b1r = b1.reshape(1, _H1)
    b2r = b2.reshape(1, _H2)
    b3r = b3.reshape(1, _H3)
    b4r = b4.reshape(1, _E)
    logits = _mlp_logits(x, W1, b1r, W2, b2r, W3, b3r, W4, b4r)
    return _route(logits.reshape(_B * _E)).reshape(_B, _E)


# speedup vs baseline: 1.9318x; 1.9318x over previous
import jax
import jax.numpy as jnp
from jax.experimental import pallas as pl
from jax.experimental.pallas import tpu as pltpu

_B, _D, _E = 8192, 4096, 64
_H1, _H2, _H3 = 128, 256, 128
_R = 1024
_N = _B // _R

_DN = (((1,), (1,)), ((), ()))


def _gating_kernel(x_ref, w1_ref, b1_ref, w2_ref, b2_ref, w3_ref, b3_ref,
                   w4_ref, b4_ref, out_ref, acc_ref):
    step = pl.program_id(0)

    x = x_ref[...]
    h = jax.lax.dot_general(x, w1_ref[...], _DN,
                            preferred_element_type=jnp.float32) + b1_ref[...]
    h = jnp.maximum(h, 0.0)
    h = jax.lax.dot_general(h, w2_ref[...], _DN,
                            preferred_element_type=jnp.float32) + b2_ref[...]
    h = jnp.where(h >= 0, h, 0.01 * h)
    h = jax.lax.dot_general(h, w3_ref[...], _DN,
                            preferred_element_type=jnp.float32) + b3_ref[...]
    h = jnp.where(h >= 0, h, 0.01 * h)
    logits = jax.lax.dot_general(h, w4_ref[...], _DN,
                                 preferred_element_type=jnp.float32) + b4_ref[...]

    col = jax.lax.broadcasted_iota(jnp.int32, (_R, _E), 1)
    m1 = jnp.max(logits, axis=1, keepdims=True)
    i1 = jnp.min(jnp.where(logits == m1, col, _E), axis=1, keepdims=True)
    masked = jnp.where(col == i1, -jnp.inf, logits)
    m2 = jnp.max(masked, axis=1, keepdims=True)
    i2 = jnp.min(jnp.where(masked == m2, col, _E), axis=1, keepdims=True)

    psum = jnp.sum(m1) + jnp.sum(m2)
    prev = jnp.where(step == 0, 0.0, acc_ref[0])
    total = prev + psum
    acc_ref[0] = total

    out_ref[...] = jnp.zeros((_R, _E), jnp.float32)

    @pl.when(step == _N - 1)
    def _write_row0():
        lane = jax.lax.broadcasted_iota(jnp.int32, (1, _E), 1)
        row = (jnp.where(lane == i1[0:1], m1[0:1] / total, 0.0)
               + jnp.where(lane == i2[0:1], m2[0:1] / total, 0.0))
        out_ref[0:1, :] = row


def kernel(x, W1, b1, W2, b2, W3, b3, W4, b4):
    b1r = b1.reshape(1, _H1)
    b2r = b2.reshape(1, _H2)
    b3r = b3.reshape(1, _H3)
    b4r = b4.reshape(1, _E)
    rev = lambda i: (_N - 1 - i, 0)
    fixed = lambda i: (0, 0)
    return pl.pallas_call(
        _gating_kernel,
        grid=(_N,),
        in_specs=[
            pl.BlockSpec((_R, _D), rev),
            pl.BlockSpec((_H1, _D), fixed),
            pl.BlockSpec((1, _H1), fixed),
            pl.BlockSpec((_H2, _H1), fixed),
            pl.BlockSpec((1, _H2), fixed),
            pl.BlockSpec((_H3, _H2), fixed),
            pl.BlockSpec((1, _H3), fixed),
            pl.BlockSpec((_E, _H3), fixed),
            pl.BlockSpec((1, _E), fixed),
        ],
        out_specs=pl.BlockSpec((_R, _E), rev),
        out_shape=jax.ShapeDtypeStruct((_B, _E), jnp.float32),
        scratch_shapes=[pltpu.SMEM((1,), jnp.float32)],
    )(x, W1, b1r, W2, b2r, W3, b3r, W4, b4r)


# speedup vs baseline: 1.9615x; 1.0154x over previous
import jax
import jax.numpy as jnp
from jax.experimental import pallas as pl
from jax.experimental.pallas import tpu as pltpu

_B, _D, _E = 8192, 4096, 64
_H1, _H2, _H3 = 128, 256, 128
_R = 1024
_N = _B // _R
_DH = _D // 2

_DN = (((1,), (1,)), ((), ()))


def _gating_kernel(xl_ref, xr_ref, w1_ref, b1_ref, w2_ref, b2_ref, w3_ref,
                   b3_ref, w4_ref, b4_ref, out_ref, acc_ref):
    step = pl.program_id(0)

    h = (jax.lax.dot_general(xl_ref[...], w1_ref[:, :_DH], _DN,
                             preferred_element_type=jnp.float32)
         + jax.lax.dot_general(xr_ref[...], w1_ref[:, _DH:], _DN,
                               preferred_element_type=jnp.float32)
         + b1_ref[...])
    h = jnp.maximum(h, 0.0)
    h = jax.lax.dot_general(h, w2_ref[...], _DN,
                            preferred_element_type=jnp.float32) + b2_ref[...]
    h = jnp.where(h >= 0, h, 0.01 * h)
    h = jax.lax.dot_general(h, w3_ref[...], _DN,
                            preferred_element_type=jnp.float32) + b3_ref[...]
    h = jnp.where(h >= 0, h, 0.01 * h)
    logits = jax.lax.dot_general(h, w4_ref[...], _DN,
                                 preferred_element_type=jnp.float32) + b4_ref[...]

    col = jax.lax.broadcasted_iota(jnp.int32, (_R, _E), 1)
    m1 = jnp.max(logits, axis=1, keepdims=True)
    i1 = jnp.min(jnp.where(logits == m1, col, _E), axis=1, keepdims=True)
    masked = jnp.where(col == i1, -jnp.inf, logits)
    m2 = jnp.max(masked, axis=1, keepdims=True)
    i2 = jnp.min(jnp.where(masked == m2, col, _E), axis=1, keepdims=True)

    psum = jnp.sum(m1) + jnp.sum(m2)
    prev = jnp.where(step == 0, 0.0, acc_ref[0])
    total = prev + psum
    acc_ref[0] = total

    out_ref[...] = jnp.zeros((_R, _E), jnp.float32)

    @pl.when(step == _N - 1)
    def _write_row0():
        lane = jax.lax.broadcasted_iota(jnp.int32, (1, _E), 1)
        row = (jnp.where(lane == i1[0:1], m1[0:1] / total, 0.0)
               + jnp.where(lane == i2[0:1], m2[0:1] / total, 0.0))
        out_ref[0:1, :] = row


def kernel(x, W1, b1, W2, b2, W3, b3, W4, b4):
    b1r = b1.reshape(1, _H1)
    b2r = b2.reshape(1, _H2)
    b3r = b3.reshape(1, _H3)
    b4r = b4.reshape(1, _E)
    revl = lambda i: (_N - 1 - i, 0)
    revr = lambda i: (_N - 1 - i, 1)
    fixed = lambda i: (0, 0)
    return pl.pallas_call(
        _gating_kernel,
        grid=(_N,),
        in_specs=[
            pl.BlockSpec((_R, _DH), revl),
            pl.BlockSpec((_R, _DH), revr),
            pl.BlockSpec((_H1, _D), fixed),
            pl.BlockSpec((1, _H1), fixed),
            pl.BlockSpec((_H2, _H1), fixed),
            pl.BlockSpec((1, _H2), fixed),
            pl.BlockSpec((_H3, _H2), fixed),
            pl.BlockSpec((1, _H3), fixed),
            pl.BlockSpec((_E, _H3), fixed),
            pl.BlockSpec((1, _E), fixed),
        ],
        out_specs=pl.BlockSpec((_R, _E), revl),
        out_shape=jax.ShapeDtypeStruct((_B, _E), jnp.float32),
        scratch_shapes=[pltpu.SMEM((1,), jnp.float32)],
    )(x, x, W1, b1r, W2, b2r, W3, b3r, W4, b4r)
